# baseline (speedup 1.0000x reference)
import functools

import jax
import jax.numpy as jnp
import numpy as np
from jax import lax
from jax.experimental import pallas as pl
from jax.experimental.pallas import tpu as pltpu
from jax.experimental.pallas import tpu_sc as plsc

NUM_CHANNEL = 4
NGRAM_SIZE = 3
LEVELS = 100
DIM = 10000
SEQ_LEN = 256
CHOSEN_FEAT = [547, 548, 549, 551, 554, 556, 557, 558, 559, 560, 561, 562,
               563, 565, 566, 567, 570, 576, 580, 581, 582, 583, 584, 585,
               588, 593, 598, 599, 600]


NW = 32
WCOLS = 320
WIN = 336
SCPAD = (NW - 1) * WCOLS + WIN
NCH = WIN // 16
NQ = WCOLS // 16


def _sc_body(ltpad_hbm, keyspad_hbm, sig_hbm, out_hbm,
             tbl_v, keys_v, sig_v, ch_v, samp_v, idx_s):
    wid = lax.axis_index("s") * 2 + lax.axis_index("c")
    base = wid * WCOLS

    pltpu.sync_copy(ltpad_hbm.at[:, pl.ds(base, WIN)], tbl_v)
    pltpu.sync_copy(keyspad_hbm.at[:, pl.ds(base, WIN)], keys_v)
    pltpu.sync_copy(sig_hbm, sig_v)

    lane = lax.iota(jnp.int32, 16)

    @plsc.parallel_loop(0, SEQ_LEN * NUM_CHANNEL // 16)
    def body_idx(jj):
        s = sig_v[pl.ds(jj * 16, 16)]
        iv = jnp.clip((s * LEVELS).astype(jnp.int32), 0, LEVELS - 1)
        for k in range(16):
            idx_s[jj * 16 + k] = jnp.max(jnp.where(lane == k, iv, -1))

    for j in range(NCH):
        kv = [keys_v[c, pl.ds(j * 16, 16)] for c in range(4)]

        @plsc.parallel_loop(0, SEQ_LEN, unroll=8)
        def body_a(t, j=j, kv=kv):
            acc = tbl_v[idx_s[4 * t + 0], pl.ds(j * 16, 16)] * kv[0]
            acc = acc + tbl_v[idx_s[4 * t + 1], pl.ds(j * 16, 16)] * kv[1]
            acc = acc + tbl_v[idx_s[4 * t + 2], pl.ds(j * 16, 16)] * kv[2]
            acc = acc + tbl_v[idx_s[4 * t + 3], pl.ds(j * 16, 16)] * kv[3]
            ch_v[t, pl.ds(j * 16, 16)] = acc

    for q in range(NQ):
        @plsc.parallel_loop(0, SEQ_LEN - NGRAM_SIZE + 1, unroll=8,
                            carry=jnp.zeros((16,), jnp.float32))
        def body_b(t, acc, q=q):
            g = (ch_v[t, pl.ds(q * 16, 16)]
                 * ch_v[t + 1, pl.ds(q * 16 + 1, 16)]
                 * ch_v[t + 2, pl.ds(q * 16 + 2, 16)])
            return acc + g
        samp_v[pl.ds(q * 16, 16)] = body_b

    pltpu.sync_copy(samp_v, out_hbm.at[pl.ds(base, WCOLS)])


def _sc_sample_hv(signals, keys, level_table):
    ltpad = jnp.concatenate(
        [level_table[:, -2:], level_table,
         jnp.zeros((LEVELS, SCPAD - DIM - 2), level_table.dtype)], axis=1)
    kpad = jnp.concatenate(
        [keys[:, -2:], keys,
         jnp.zeros((NUM_CHANNEL, SCPAD - DIM - 2), keys.dtype)], axis=1)
    mesh = plsc.VectorSubcoreMesh(core_axis_name="c", subcore_axis_name="s")
    return pl.kernel(
        _sc_body, mesh=mesh,
        compiler_params=pltpu.CompilerParams(use_tc_tiling_on_sc=False,
                                             needs_layout_passes=False),
        out_type=jax.ShapeDtypeStruct((NW * WCOLS,), jnp.float32),
        scratch_types=[
            pltpu.VMEM((LEVELS, WIN), jnp.float32),
            pltpu.VMEM((NUM_CHANNEL, WIN), jnp.float32),
            pltpu.VMEM((SEQ_LEN * NUM_CHANNEL,), jnp.float32),
            pltpu.VMEM((SEQ_LEN, WIN), jnp.float32),
            pltpu.VMEM((WCOLS,), jnp.float32),
            pltpu.SMEM((SEQ_LEN * NUM_CHANNEL,), jnp.int32),
        ],
    )(ltpad, kpad, signals.reshape(-1))


BLK = 2048
NBLK = pl.cdiv(DIM, BLK)


def _expr_kernel(samp_ref, vals_ref, wf_ref, bf_ref, mf_ref, wm_ref, bm_ref,
                 out_ref):
    proj = vals_ref[...] * wf_ref[...]
    fhv = jnp.cos(proj + bf_ref[...]) * jnp.sin(proj)

    wm = wm_ref[...].astype(jnp.bfloat16)
    mf = mf_ref[...].astype(jnp.bfloat16)
    mrows = []
    for e in range(6):
        r = lax.dot_general(wm[e], mf[e][:, None],
                            (((1,), (0,)), ((), ())),
                            preferred_element_type=jnp.float32)
        mrows.append(r[:, 0])
    mproj = jnp.stack(mrows)
    mhv = jnp.cos(mproj + bm_ref[...]) * jnp.sin(mproj)
    mfcc_hv = mhv[0] * mhv[1] * mhv[2] * mhv[3] * mhv[4] * mhv[5]

    f = {cf: fhv[j] for j, cf in enumerate(CHOSEN_FEAT)}
    expr = (f[547] * f[559] * f[565]
            + f[548] * f[560] * f[566]
            + f[549] * f[561] * f[567]
            + f[551] * f[554]
            + f[556] * f[558] * f[584] * f[557] * f[585] * f[581] * f[580]
            * f[582] * f[583] * f[598] * f[600] * f[599]
            + f[562] + f[563]
            + f[570] * f[588]
            + f[576] + f[593]
            + mfcc_hv)
    out = samp_ref[0, :] * expr
    out_ref[0, :] = jnp.where(out > 0, 1.0, -1.0)


def _expr_part(sample, feat, W_feat, b_feat, W_mfcc, b_mfcc):
    sel = np.array([cf - 1 for cf in CHOSEN_FEAT])
    vals = feat[sel][:, None]
    mf = feat[: 6 * 91].reshape(6, 91)
    return pl.pallas_call(
        _expr_kernel,
        grid=(NBLK,),
        in_specs=[
            pl.BlockSpec((1, BLK), lambda i: (0, i)),
            pl.BlockSpec((len(CHOSEN_FEAT), 1), lambda i: (0, 0)),
            pl.BlockSpec((len(CHOSEN_FEAT), BLK), lambda i: (0, i)),
            pl.BlockSpec((len(CHOSEN_FEAT), BLK), lambda i: (0, i)),
            pl.BlockSpec((6, 91), lambda i: (0, 0)),
            pl.BlockSpec((6, BLK, 91), lambda i: (0, i, 0)),
            pl.BlockSpec((6, BLK), lambda i: (0, i)),
        ],
        out_specs=pl.BlockSpec((1, BLK), lambda i: (0, i)),
        out_shape=jax.ShapeDtypeStruct((1, DIM), jnp.float32),
    )(sample.reshape(1, -1)[:, :DIM], vals, W_feat, b_feat, mf,
      W_mfcc, b_mfcc)


@jax.jit
def _run(signals, feat, keys, level_table, W_feat, b_feat, W_mfcc, b_mfcc):
    sample = _sc_sample_hv(signals, keys, level_table)
    out = _expr_part(sample, feat, W_feat, b_feat, W_mfcc, b_mfcc)
    return out.reshape(-1)


def kernel(signals, feat, keys, level_table, W_feat, b_feat, W_mfcc, b_mfcc):
    return _run(signals, feat, keys, level_table, W_feat, b_feat,
                W_mfcc, b_mfcc)


# speedup vs baseline: 1.0306x; 1.0306x over previous
import functools

import jax
import jax.numpy as jnp
import numpy as np
from jax import lax
from jax.experimental import pallas as pl
from jax.experimental.pallas import tpu as pltpu
from jax.experimental.pallas import tpu_sc as plsc

NUM_CHANNEL = 4
NGRAM_SIZE = 3
LEVELS = 100
DIM = 10000
SEQ_LEN = 256
CHOSEN_FEAT = [547, 548, 549, 551, 554, 556, 557, 558, 559, 560, 561, 562,
               563, 565, 566, 567, 570, 576, 580, 581, 582, 583, 584, 585,
               588, 593, 598, 599, 600]


NW = 32
WCOLS = 320
WIN = 336
SCPAD = (NW - 1) * WCOLS + WIN
NCH = WIN // 16
NQ = WCOLS // 16


def _sc_body(ltpad_hbm, keyspad_hbm, sig_hbm, out_hbm,
             tbl_v, keys_v, sig_v, ch_v, samp_v, idx_s):
    wid = lax.axis_index("s") * 2 + lax.axis_index("c")
    base = wid * WCOLS

    pltpu.sync_copy(ltpad_hbm.at[:, pl.ds(base, WIN)], tbl_v)
    pltpu.sync_copy(keyspad_hbm.at[:, pl.ds(base, WIN)], keys_v)
    pltpu.sync_copy(sig_hbm, sig_v)

    lane = lax.iota(jnp.int32, 16)

    @plsc.parallel_loop(0, SEQ_LEN * NUM_CHANNEL // 16)
    def body_idx(jj):
        s = sig_v[pl.ds(jj * 16, 16)]
        iv = jnp.clip((s * LEVELS).astype(jnp.int32), 0, LEVELS - 1)
        for k in range(16):
            idx_s[jj * 16 + k] = jnp.max(jnp.where(lane == k, iv, -1))

    for j in range(NCH):
        kv = [keys_v[c, pl.ds(j * 16, 16)] for c in range(4)]

        @plsc.parallel_loop(0, SEQ_LEN, unroll=4)
        def body_a(t, j=j, kv=kv):
            acc = tbl_v[idx_s[4 * t + 0], pl.ds(j * 16, 16)] * kv[0]
            acc = acc + tbl_v[idx_s[4 * t + 1], pl.ds(j * 16, 16)] * kv[1]
            acc = acc + tbl_v[idx_s[4 * t + 2], pl.ds(j * 16, 16)] * kv[2]
            acc = acc + tbl_v[idx_s[4 * t + 3], pl.ds(j * 16, 16)] * kv[3]
            ch_v[t, pl.ds(j * 16, 16)] = acc

    for q in range(NQ):
        @plsc.parallel_loop(0, SEQ_LEN - NGRAM_SIZE + 1, unroll=4,
                            carry=jnp.zeros((16,), jnp.float32))
        def body_b(t, acc, q=q):
            g = (ch_v[t, pl.ds(q * 16, 16)]
                 * ch_v[t + 1, pl.ds(q * 16 + 1, 16)]
                 * ch_v[t + 2, pl.ds(q * 16 + 2, 16)])
            return acc + g
        samp_v[pl.ds(q * 16, 16)] = body_b

    pltpu.sync_copy(samp_v, out_hbm.at[pl.ds(base, WCOLS)])


def _sc_sample_hv(signals, keys, level_table):
    ltpad = jnp.concatenate(
        [level_table[:, -2:], level_table,
         jnp.zeros((LEVELS, SCPAD - DIM - 2), level_table.dtype)], axis=1)
    kpad = jnp.concatenate(
        [keys[:, -2:], keys,
         jnp.zeros((NUM_CHANNEL, SCPAD - DIM - 2), keys.dtype)], axis=1)
    mesh = plsc.VectorSubcoreMesh(core_axis_name="c", subcore_axis_name="s")
    return pl.kernel(
        _sc_body, mesh=mesh,
        compiler_params=pltpu.CompilerParams(use_tc_tiling_on_sc=False,
                                             needs_layout_passes=False),
        out_type=jax.ShapeDtypeStruct((NW * WCOLS,), jnp.float32),
        scratch_types=[
            pltpu.VMEM((LEVELS, WIN), jnp.float32),
            pltpu.VMEM((NUM_CHANNEL, WIN), jnp.float32),
            pltpu.VMEM((SEQ_LEN * NUM_CHANNEL,), jnp.float32),
            pltpu.VMEM((SEQ_LEN, WIN), jnp.float32),
            pltpu.VMEM((WCOLS,), jnp.float32),
            pltpu.SMEM((SEQ_LEN * NUM_CHANNEL,), jnp.int32),
        ],
    )(ltpad, kpad, signals.reshape(-1))


BLK = 2048
NBLK = pl.cdiv(DIM, BLK)


def _expr_kernel(samp_ref, vals_ref, wf_ref, bf_ref, mf_ref, wm_ref, bm_ref,
                 out_ref):
    proj = vals_ref[...] * wf_ref[...]
    fhv = jnp.cos(proj + bf_ref[...]) * jnp.sin(proj)

    wm = wm_ref[...].astype(jnp.bfloat16)
    mf = mf_ref[...].astype(jnp.bfloat16)
    mrows = []
    for e in range(6):
        r = lax.dot_general(wm[e], mf[e][:, None],
                            (((1,), (0,)), ((), ())),
                            preferred_element_type=jnp.float32)
        mrows.append(r[:, 0])
    mproj = jnp.stack(mrows)
    mhv = jnp.cos(mproj + bm_ref[...]) * jnp.sin(mproj)
    mfcc_hv = mhv[0] * mhv[1] * mhv[2] * mhv[3] * mhv[4] * mhv[5]

    f = {cf: fhv[j] for j, cf in enumerate(CHOSEN_FEAT)}
    expr = (f[547] * f[559] * f[565]
            + f[548] * f[560] * f[566]
            + f[549] * f[561] * f[567]
            + f[551] * f[554]
            + f[556] * f[558] * f[584] * f[557] * f[585] * f[581] * f[580]
            * f[582] * f[583] * f[598] * f[600] * f[599]
            + f[562] + f[563]
            + f[570] * f[588]
            + f[576] + f[593]
            + mfcc_hv)
    out = samp_ref[0, :] * expr
    out_ref[0, :] = jnp.where(out > 0, 1.0, -1.0)


def _expr_part(sample, feat, W_feat, b_feat, W_mfcc, b_mfcc):
    sel = np.array([cf - 1 for cf in CHOSEN_FEAT])
    vals = feat[sel][:, None]
    mf = feat[: 6 * 91].reshape(6, 91)
    return pl.pallas_call(
        _expr_kernel,
        grid=(NBLK,),
        in_specs=[
            pl.BlockSpec((1, BLK), lambda i: (0, i)),
            pl.BlockSpec((len(CHOSEN_FEAT), 1), lambda i: (0, 0)),
            pl.BlockSpec((len(CHOSEN_FEAT), BLK), lambda i: (0, i)),
            pl.BlockSpec((len(CHOSEN_FEAT), BLK), lambda i: (0, i)),
            pl.BlockSpec((6, 91), lambda i: (0, 0)),
            pl.BlockSpec((6, BLK, 91), lambda i: (0, i, 0)),
            pl.BlockSpec((6, BLK), lambda i: (0, i)),
        ],
        out_specs=pl.BlockSpec((1, BLK), lambda i: (0, i)),
        out_shape=jax.ShapeDtypeStruct((1, DIM), jnp.float32),
    )(sample.reshape(1, -1)[:, :DIM], vals, W_feat, b_feat, mf,
      W_mfcc, b_mfcc)


@jax.jit
def _run(signals, feat, keys, level_table, W_feat, b_feat, W_mfcc, b_mfcc):
    sample = _sc_sample_hv(signals, keys, level_table)
    out = _expr_part(sample, feat, W_feat, b_feat, W_mfcc, b_mfcc)
    return out.reshape(-1)


def kernel(signals, feat, keys, level_table, W_feat, b_feat, W_mfcc, b_mfcc):
    return _run(signals, feat, keys, level_table, W_feat, b_feat,
                W_mfcc, b_mfcc)


# speedup vs baseline: 1.1764x; 1.1415x over previous
import functools

import jax
import jax.numpy as jnp
import numpy as np
from jax import lax
from jax.experimental import pallas as pl
from jax.experimental.pallas import tpu as pltpu
from jax.experimental.pallas import tpu_sc as plsc

NUM_CHANNEL = 4
NGRAM_SIZE = 3
LEVELS = 100
DIM = 10000
SEQ_LEN = 256
CHOSEN_FEAT = [547, 548, 549, 551, 554, 556, 557, 558, 559, 560, 561, 562,
               563, 565, 566, 567, 570, 576, 580, 581, 582, 583, 584, 585,
               588, 593, 598, 599, 600]


NW = 32
WCOLS = 320
WIN = 336
SCPAD = (NW - 1) * WCOLS + WIN
NCH = WIN // 16
NQ = WCOLS // 16


def _sc_body(ltpad_hbm, keyspad_hbm, sig_hbm, out_hbm,
             tbl_v, keys_v, sig_v, ch_v, samp_v, idx_s):
    wid = lax.axis_index("s") * 2 + lax.axis_index("c")
    base = wid * WCOLS

    pltpu.sync_copy(ltpad_hbm.at[:, pl.ds(base, WIN)], tbl_v)
    pltpu.sync_copy(keyspad_hbm.at[:, pl.ds(base, WIN)], keys_v)
    pltpu.sync_copy(sig_hbm, sig_v)

    lane = lax.iota(jnp.int32, 16)

    @plsc.parallel_loop(0, SEQ_LEN * NUM_CHANNEL // 16)
    def body_idx(jj):
        s = sig_v[pl.ds(jj * 16, 16)]
        iv = jnp.clip((s * LEVELS).astype(jnp.int32), 0, LEVELS - 1)
        for k in range(16):
            idx_s[jj * 16 + k] = jnp.max(jnp.where(lane == k, iv, -1))

    for j in range(NCH):
        kv = [keys_v[c, pl.ds(j * 16, 16)] for c in range(4)]

        @plsc.parallel_loop(0, SEQ_LEN, unroll=4)
        def body_a(t, j=j, kv=kv):
            acc = tbl_v[idx_s[4 * t + 0], pl.ds(j * 16, 16)] * kv[0]
            acc = acc + tbl_v[idx_s[4 * t + 1], pl.ds(j * 16, 16)] * kv[1]
            acc = acc + tbl_v[idx_s[4 * t + 2], pl.ds(j * 16, 16)] * kv[2]
            acc = acc + tbl_v[idx_s[4 * t + 3], pl.ds(j * 16, 16)] * kv[3]
            ch_v[t, pl.ds(j * 16, 16)] = acc

    for q in range(NQ):
        @plsc.parallel_loop(0, SEQ_LEN - NGRAM_SIZE + 1, unroll=4,
                            carry=jnp.zeros((16,), jnp.float32))
        def body_b(t, acc, q=q):
            g = (ch_v[t, pl.ds(q * 16, 16)]
                 * ch_v[t + 1, pl.ds(q * 16 + 1, 16)]
                 * ch_v[t + 2, pl.ds(q * 16 + 2, 16)])
            return acc + g
        samp_v[pl.ds(q * 16, 16)] = body_b

    pltpu.sync_copy(samp_v, out_hbm.at[pl.ds(base, WCOLS)])


def _sc_sample_hv(signals, keys, level_table):
    ltpad = jnp.concatenate(
        [level_table[:, -2:], level_table,
         jnp.zeros((LEVELS, SCPAD - DIM - 2), level_table.dtype)], axis=1)
    kpad = jnp.concatenate(
        [keys[:, -2:], keys,
         jnp.zeros((NUM_CHANNEL, SCPAD - DIM - 2), keys.dtype)], axis=1)
    mesh = plsc.VectorSubcoreMesh(core_axis_name="c", subcore_axis_name="s")
    return pl.kernel(
        _sc_body, mesh=mesh,
        compiler_params=pltpu.CompilerParams(use_tc_tiling_on_sc=False,
                                             needs_layout_passes=False),
        out_type=jax.ShapeDtypeStruct((NW * WCOLS,), jnp.float32),
        scratch_types=[
            pltpu.VMEM((LEVELS, WIN), jnp.float32),
            pltpu.VMEM((NUM_CHANNEL, WIN), jnp.float32),
            pltpu.VMEM((SEQ_LEN * NUM_CHANNEL,), jnp.float32),
            pltpu.VMEM((SEQ_LEN, WIN), jnp.float32),
            pltpu.VMEM((WCOLS,), jnp.float32),
            pltpu.SMEM((SEQ_LEN * NUM_CHANNEL,), jnp.int32),
        ],
    )(ltpad, kpad, signals.reshape(-1))


BLK = 2048
NBLK = pl.cdiv(DIM, BLK)


def _expr_kernel(vals_ref, wf_ref, bf_ref, mf_ref, wm_ref, bm_ref, out_ref):
    proj = vals_ref[...] * wf_ref[...]
    fhv = jnp.cos(proj + bf_ref[...]) * jnp.sin(proj)

    wm = wm_ref[...].astype(jnp.bfloat16)
    mf = mf_ref[...].astype(jnp.bfloat16)
    mrows = []
    for e in range(6):
        r = lax.dot_general(wm[e], mf[e][:, None],
                            (((1,), (0,)), ((), ())),
                            preferred_element_type=jnp.float32)
        mrows.append(r[:, 0])
    mproj = jnp.stack(mrows)
    mhv = jnp.cos(mproj + bm_ref[...]) * jnp.sin(mproj)
    mfcc_hv = mhv[0] * mhv[1] * mhv[2] * mhv[3] * mhv[4] * mhv[5]

    f = {cf: fhv[j] for j, cf in enumerate(CHOSEN_FEAT)}
    expr = (f[547] * f[559] * f[565]
            + f[548] * f[560] * f[566]
            + f[549] * f[561] * f[567]
            + f[551] * f[554]
            + f[556] * f[558] * f[584] * f[557] * f[585] * f[581] * f[580]
            * f[582] * f[583] * f[598] * f[600] * f[599]
            + f[562] + f[563]
            + f[570] * f[588]
            + f[576] + f[593]
            + mfcc_hv)
    out_ref[0, :] = expr


def _expr_part(feat, W_feat, b_feat, W_mfcc, b_mfcc):
    sel = np.array([cf - 1 for cf in CHOSEN_FEAT])
    vals = feat[sel][:, None]
    mf = feat[: 6 * 91].reshape(6, 91)
    return pl.pallas_call(
        _expr_kernel,
        grid=(NBLK,),
        in_specs=[
            pl.BlockSpec((len(CHOSEN_FEAT), 1), lambda i: (0, 0)),
            pl.BlockSpec((len(CHOSEN_FEAT), BLK), lambda i: (0, i)),
            pl.BlockSpec((len(CHOSEN_FEAT), BLK), lambda i: (0, i)),
            pl.BlockSpec((6, 91), lambda i: (0, 0)),
            pl.BlockSpec((6, BLK, 91), lambda i: (0, i, 0)),
            pl.BlockSpec((6, BLK), lambda i: (0, i)),
        ],
        out_specs=pl.BlockSpec((1, BLK), lambda i: (0, i)),
        out_shape=jax.ShapeDtypeStruct((1, DIM), jnp.float32),
    )(vals, W_feat, b_feat, mf, W_mfcc, b_mfcc)


def _combine_kernel(samp_ref, expr_ref, out_ref):
    out = samp_ref[0, :DIM] * expr_ref[0, :]
    out_ref[0, :] = jnp.where(out > 0, 1.0, -1.0)


def _combine(sample, expr):
    return pl.pallas_call(
        _combine_kernel,
        out_shape=jax.ShapeDtypeStruct((1, DIM), jnp.float32),
    )(sample.reshape(1, -1), expr)


@jax.jit
def _run(signals, feat, keys, level_table, W_feat, b_feat, W_mfcc, b_mfcc):
    expr = _expr_part(feat, W_feat, b_feat, W_mfcc, b_mfcc)
    sample = _sc_sample_hv(signals, keys, level_table)
    return _combine(sample, expr).reshape(-1)


def kernel(signals, feat, keys, level_table, W_feat, b_feat, W_mfcc, b_mfcc):
    return _run(signals, feat, keys, level_table, W_feat, b_feat,
                W_mfcc, b_mfcc)
